# Initial kernel scaffold; baseline (speedup 1.0000x reference)
import jax
import jax.numpy as jnp
from jax.experimental import pallas as pl


def kernel(x, token_table, pos_table):
    raise NotImplementedError("write your pallas kernel here")


# speedup vs baseline: 2.6060x; 2.6060x over previous
import functools

import jax
import jax.numpy as jnp
from jax import lax
from jax.experimental import pallas as pl
from jax.experimental.pallas import tpu as pltpu
from jax.experimental.pallas import tpu_sc as plsc


@functools.lru_cache(maxsize=None)
def _build(N, V, D, L):
    info = plsc.get_sparse_core_info()
    NC, NS = info.num_cores, info.num_subcores
    NW = NC * NS
    LANES = info.num_lanes

    rows_per_w = N // NW
    assert rows_per_w * NW == N
    assert rows_per_w % L == 0
    CHUNK = 2 * L
    if rows_per_w % CHUNK != 0:
        CHUNK = L
    n_chunks = rows_per_w // CHUNK
    G = 5
    GC = CHUNK // G
    assert GC <= 128 and GC % 8 == 0
    assert D % LANES == 0

    mesh = plsc.VectorSubcoreMesh(core_axis_name="c", subcore_axis_name="s")

    @functools.partial(
        pl.kernel,
        mesh=mesh,
        out_type=jax.ShapeDtypeStruct((N, D), jnp.float32),
        compiler_params=pltpu.CompilerParams(use_tc_tiling_on_sc=False),
        scratch_types=[
            pltpu.VMEM((G, GC), jnp.int32),
            pltpu.VMEM((CHUNK, D), jnp.float32),
            pltpu.VMEM((CHUNK, D), jnp.float32),
            pltpu.SemaphoreType.DMA,
        ],
    )
    def emb_kernel(x_hbm, tok_hbm, pos_hbm, out_hbm, idx_v, rows_v, pos_v, sem):
        wid = lax.axis_index("s") * NC + lax.axis_index("c")
        wbase = wid * rows_per_w

        for r in range(CHUNK // L):
            pltpu.sync_copy(pos_hbm, pos_v.at[pl.ds(r * L, L)])

        def chunk_body(c, carry):
            base = wbase + c * CHUNK
            for j in range(G):
                pltpu.sync_copy(x_hbm.at[pl.ds(base + j * GC, GC)], idx_v.at[j])
            for j in range(G):
                pltpu.async_copy(
                    tok_hbm.at[idx_v.at[j]],
                    rows_v.at[pl.ds(j * GC, GC)],
                    sem,
                ).wait()

            def add_rows(r, c2):
                for u in range(4):
                    row = r * 4 + u
                    for d in range(D // LANES):
                        sl = pl.ds(d * LANES, LANES)
                        plsc.addupdate(rows_v.at[row, sl], pos_v[row, sl])
                return c2

            lax.fori_loop(0, CHUNK // 4, add_rows, 0)
            pltpu.sync_copy(rows_v, out_hbm.at[pl.ds(base, CHUNK)])
            return carry

        lax.fori_loop(0, n_chunks, chunk_body, 0)

    return emb_kernel


def kernel(x, token_table, pos_table):
    B, L = x.shape
    V, D = token_table.shape
    xf = x.reshape(B * L).astype(jnp.int32)
    emb = _build(B * L, V, D, L)
    out = emb(xf, token_table, pos_table)
    return out.reshape(B, L, D)


# speedup vs baseline: 3.4869x; 1.3380x over previous
import functools

import jax
import jax.numpy as jnp
from jax import lax
from jax.experimental import pallas as pl
from jax.experimental.pallas import tpu as pltpu
from jax.experimental.pallas import tpu_sc as plsc


@functools.lru_cache(maxsize=None)
def _build(N, V, D, L):
    info = plsc.get_sparse_core_info()
    NC, NS = info.num_cores, info.num_subcores
    NW = NC * NS
    LANES = info.num_lanes

    rows_per_w = N // NW
    assert rows_per_w * NW == N
    assert rows_per_w % L == 0
    CHUNK = 2 * L
    if rows_per_w % CHUNK != 0:
        CHUNK = L
    n_chunks = rows_per_w // CHUNK
    assert n_chunks % 2 == 0
    G = 5
    GC = CHUNK // G
    assert GC <= 128 and GC % 8 == 0
    assert D % LANES == 0

    mesh = plsc.VectorSubcoreMesh(core_axis_name="c", subcore_axis_name="s")

    @functools.partial(
        pl.kernel,
        mesh=mesh,
        out_type=jax.ShapeDtypeStruct((N, D), jnp.float32),
        compiler_params=pltpu.CompilerParams(use_tc_tiling_on_sc=False),
        scratch_types=[
            pltpu.VMEM((2, G, GC), jnp.int32),
            pltpu.VMEM((CHUNK, D), jnp.float32),
            pltpu.VMEM((CHUNK, D), jnp.float32),
            pltpu.VMEM((L, D), jnp.float32),
            pltpu.SemaphoreType.DMA,
            pltpu.SemaphoreType.DMA,
            pltpu.SemaphoreType.DMA,
            pltpu.SemaphoreType.DMA,
        ],
    )
    def emb_kernel(x_hbm, tok_hbm, pos_hbm, out_hbm,
                   idx_v, rows0, rows1, pos_v, g0, g1, o0, o1):
        rows = (rows0, rows1)
        gsem = (g0, g1)
        osem = (o0, o1)
        wid = lax.axis_index("s") * NC + lax.axis_index("c")
        wbase = wid * rows_per_w

        pltpu.sync_copy(pos_hbm, pos_v)

        def issue(c, b):
            base = wbase + c * CHUNK
            for j in range(G):
                pltpu.sync_copy(x_hbm.at[pl.ds(base + j * GC, GC)],
                                idx_v.at[b, j])
            for j in range(G):
                pltpu.async_copy(tok_hbm.at[idx_v.at[b, j]],
                                 rows[b].at[pl.ds(j * GC, GC)], gsem[b])

        def drain_gathers(b):
            for j in range(G):
                pltpu.make_async_copy(tok_hbm.at[idx_v.at[b, j]],
                                      rows[b].at[pl.ds(j * GC, GC)],
                                      gsem[b]).wait()

        def add_pos(b):
            @plsc.parallel_loop(0, L, unroll=4)
            def _(p):
                for d in range(D // LANES):
                    sl = pl.ds(d * LANES, LANES)
                    pv = pos_v[p, sl]
                    for r in range(CHUNK // L):
                        plsc.addupdate(rows[b].at[p + r * L, sl], pv)

        def wait_out(b):
            pltpu.make_async_copy(rows[b],
                                  out_hbm.at[pl.ds(wbase, CHUNK)],
                                  osem[b]).wait()

        issue(0, 0)

        def pair_body(i, carry):
            for b in range(2):
                c = i * 2 + b

                @pl.when(c >= 1)
                def _():
                    wait_out(1 - b)

                @pl.when(c + 1 < n_chunks)
                def _():
                    issue(c + 1, 1 - b)

                drain_gathers(b)
                add_pos(b)
                pltpu.async_copy(rows[b],
                                 out_hbm.at[pl.ds(wbase + c * CHUNK, CHUNK)],
                                 osem[b])
            return carry

        lax.fori_loop(0, n_chunks // 2, pair_body, 0)
        wait_out((n_chunks - 1) % 2)

    return emb_kernel


def kernel(x, token_table, pos_table):
    B, L = x.shape
    V, D = token_table.shape
    xf = x.reshape(B * L).astype(jnp.int32)
    emb = _build(B * L, V, D, L)
    out = emb(xf, token_table, pos_table)
    return out.reshape(B, L, D)


# speedup vs baseline: 4.1746x; 1.1972x over previous
import functools

import jax
import jax.numpy as jnp
from jax import lax
from jax.experimental import pallas as pl
from jax.experimental.pallas import tpu as pltpu
from jax.experimental.pallas import tpu_sc as plsc


@functools.lru_cache(maxsize=None)
def _build(N, V, D, L):
    info = plsc.get_sparse_core_info()
    NC, NS = info.num_cores, info.num_subcores
    NW = NC * NS
    LANES = info.num_lanes

    rows_per_w = N // NW
    assert rows_per_w * NW == N
    assert rows_per_w % L == 0
    CHUNK = 2 * L
    if rows_per_w % CHUNK != 0:
        CHUNK = L
    n_chunks = rows_per_w // CHUNK
    assert n_chunks % 2 == 0
    G = 5
    GC = CHUNK // G
    assert GC <= 128 and GC % 8 == 0
    assert D % LANES == 0

    mesh = plsc.VectorSubcoreMesh(core_axis_name="c", subcore_axis_name="s")

    @functools.partial(
        pl.kernel,
        mesh=mesh,
        out_type=jax.ShapeDtypeStruct((N, D), jnp.float32),
        compiler_params=pltpu.CompilerParams(use_tc_tiling_on_sc=False),
        scratch_types=[
            pltpu.VMEM((n_chunks * G, GC), jnp.int32),
            pltpu.VMEM((CHUNK, D), jnp.float32),
            pltpu.VMEM((CHUNK, D), jnp.float32),
            pltpu.VMEM((L, D), jnp.float32),
            pltpu.SemaphoreType.DMA,
            pltpu.SemaphoreType.DMA,
            pltpu.SemaphoreType.DMA,
            pltpu.SemaphoreType.DMA,
        ],
    )
    def emb_kernel(x_hbm, tok_hbm, pos_hbm, out_hbm,
                   idx_v, rows0, rows1, pos_v, g0, g1, o0, o1):
        rows = (rows0, rows1)
        gsem = (g0, g1)
        osem = (o0, o1)
        wid = lax.axis_index("s") * NC + lax.axis_index("c")
        wbase = wid * rows_per_w

        pltpu.sync_copy(pos_hbm, pos_v)
        pltpu.sync_copy(
            x_hbm.at[pl.ds(wid * n_chunks * G, n_chunks * G)],
            idx_v,
        )

        def issue(c, b):
            for j in range(G):
                pltpu.async_copy(tok_hbm.at[idx_v.at[c * G + j]],
                                 rows[b].at[pl.ds(j * GC, GC)], gsem[b])

        def drain_gathers(c, b):
            for j in range(G):
                pltpu.make_async_copy(tok_hbm.at[idx_v.at[c * G + j]],
                                      rows[b].at[pl.ds(j * GC, GC)],
                                      gsem[b]).wait()

        def add_pos(b):
            @plsc.parallel_loop(0, L, unroll=4)
            def _(p):
                for d in range(D // LANES):
                    sl = pl.ds(d * LANES, LANES)
                    pv = pos_v[p, sl]
                    for r in range(CHUNK // L):
                        plsc.addupdate(rows[b].at[p + r * L, sl], pv)

        def wait_out(b):
            pltpu.make_async_copy(rows[b],
                                  out_hbm.at[pl.ds(wbase, CHUNK)],
                                  osem[b]).wait()

        issue(0, 0)

        def pair_body(i, carry):
            for b in range(2):
                c = i * 2 + b

                @pl.when(c >= 1)
                def _():
                    wait_out(1 - b)

                @pl.when(c + 1 < n_chunks)
                def _():
                    issue(c + 1, 1 - b)

                drain_gathers(c, b)
                add_pos(b)
                pltpu.async_copy(rows[b],
                                 out_hbm.at[pl.ds(wbase + c * CHUNK, CHUNK)],
                                 osem[b])
            return carry

        lax.fori_loop(0, n_chunks // 2, pair_body, 0)
        wait_out((n_chunks - 1) % 2)

    return emb_kernel, GC


def kernel(x, token_table, pos_table):
    B, L = x.shape
    V, D = token_table.shape
    emb, gc = _build(B * L, V, D, L)
    xf = x.reshape(B * L // gc, gc).astype(jnp.int32)
    out = emb(xf, token_table, pos_table)
    return out.reshape(B, L, D)
